# Initial kernel scaffold; baseline (speedup 1.0000x reference)
import jax
import jax.numpy as jnp
from jax.experimental import pallas as pl


def kernel(x, W1, g1, b1, W2, g2, b2, W3, g3, b3, W4, g4, b4, W5, g5, b5, L1, g6, b6, L2, Lb2, g7, b7, L3, Lb3):
    raise NotImplementedError("write your pallas kernel here")


# speedup vs baseline: 1.0007x; 1.0007x over previous
import functools

import jax
import jax.numpy as jnp
from jax.experimental import pallas as pl
from jax.experimental.pallas import tpu as pltpu

EPS = 1e-5
K = 20
INV = 1.0 / (1.0 + EPS) ** 0.5


def _knn(x, k):
    inner = -2.0 * jnp.einsum('bcn,bcm->bnm', x, x)
    xx = jnp.sum(x ** 2, axis=1, keepdims=True)
    pd = -xx - inner - jnp.transpose(xx, (0, 2, 1))
    return jax.lax.top_k(pd, k)[1]


def _graph_feature(x, k):
    B, C, N = x.shape
    idx = _knn(x, k)
    xt = jnp.transpose(x, (0, 2, 1))
    feature = xt[jnp.arange(B)[:, None, None], idx]
    central = jnp.broadcast_to(xt[:, :, None, :], (B, N, k, C))
    out = jnp.concatenate([feature - central, central], axis=3)
    return jnp.transpose(out, (0, 3, 1, 2))


def _bn(x, g, b):
    shape = [1, -1] + [1] * (x.ndim - 2)
    return x * INV * g.reshape(shape) + b.reshape(shape)


def _lrelu(x):
    return jnp.where(x >= 0, x, 0.2 * x)


def _head_kernel(h_ref, l1_ref, g6_ref, b6_ref, l2_ref, lb2_ref, g7_ref,
                 b7_ref, l3_ref, lb3_ref, out_ref):
    h = h_ref[...]
    a = jnp.dot(h, l1_ref[...].T, preferred_element_type=jnp.float32)
    a = a * INV * g6_ref[...][None, :] + b6_ref[...][None, :]
    a = jnp.where(a >= 0, a, 0.2 * a)
    b = jnp.dot(a, l2_ref[...].T, preferred_element_type=jnp.float32)
    b = b + lb2_ref[...][None, :]
    b = b * INV * g7_ref[...][None, :] + b7_ref[...][None, :]
    b = jnp.where(b >= 0, b, 0.2 * b)
    c = jnp.dot(b, l3_ref[...].T, preferred_element_type=jnp.float32)
    out_ref[...] = c + lb3_ref[...][None, :]


def _head(h, L1, g6, b6, L2, Lb2, g7, b7, L3, Lb3):
    return pl.pallas_call(
        _head_kernel,
        out_shape=jax.ShapeDtypeStruct((h.shape[0], L3.shape[0]), jnp.float32),
    )(h, L1, g6, b6, L2, Lb2, g7, b7, L3, Lb3)


def kernel(x, W1, g1, b1, W2, g2, b2, W3, g3, b3, W4, g4, b4, W5, g5, b5,
           L1, g6, b6, L2, Lb2, g7, b7, L3, Lb3):
    f = _graph_feature(x, K)
    h = _lrelu(_bn(jnp.einsum('oi,bink->bonk', W1, f), g1, b1))
    x1 = jnp.max(h, axis=-1)
    f = _graph_feature(x1, K)
    h = _lrelu(_bn(jnp.einsum('oi,bink->bonk', W2, f), g2, b2))
    x2 = jnp.max(h, axis=-1)
    f = _graph_feature(x2, K)
    h = _lrelu(_bn(jnp.einsum('oi,bink->bonk', W3, f), g3, b3))
    x3 = jnp.max(h, axis=-1)
    f = _graph_feature(x3, K)
    h = _lrelu(_bn(jnp.einsum('oi,bink->bonk', W4, f), g4, b4))
    x4 = jnp.max(h, axis=-1)
    xc = jnp.concatenate([x1, x2, x3, x4], axis=1)
    h = _lrelu(_bn(jnp.einsum('oi,bin->bon', W5, xc), g5, b5))
    p1 = jnp.max(h, axis=-1)
    p2 = jnp.mean(h, axis=-1)
    hh = jnp.concatenate([p1, p2], axis=1)
    return _head(hh, L1, g6, b6, L2, Lb2, g7, b7, L3, Lb3)


# speedup vs baseline: 13.1860x; 13.1766x over previous
import functools

import jax
import jax.numpy as jnp
from jax import lax
from jax.experimental import pallas as pl
from jax.experimental.pallas import tpu as pltpu
from jax.experimental.pallas import tpu_sc as plsc

EPS = 1e-5
K = 20
INV = float((1.0 + EPS) ** -0.5)
B = 8
N = 1024
NC, NS = 2, 16
NW = NC * NS
PPW = (B * N) // NW
WPB = N // PPW


def _lrelu(v):
    return jnp.where(v >= 0, v, 0.2 * v)


def _topk_idx(u, b):
    st = jnp.sum(u * u, axis=1)
    ub = u.astype(jnp.bfloat16)
    p = jnp.dot(ub, ub.T, preferred_element_type=jnp.float32)
    pd = 2.0 * p - st[:, None] - st[None, :]
    cols = lax.broadcasted_iota(jnp.int32, (N, N), 1)
    work = pd
    outs = []
    for _ in range(K):
        m = jnp.max(work, axis=1, keepdims=True)
        sel = jnp.where(work == m, cols, jnp.int32(N))
        a = jnp.min(sel, axis=1, keepdims=True)
        outs.append(a)
        work = jnp.where(cols == a, -jnp.inf, work)
    return jnp.concatenate(outs, axis=1) + b * N


def _edge_conv(f_ref, u, waT_ref, wbT_ref, gm_ref, bt_ref):
    ub = u.astype(jnp.bfloat16)
    bterm = jnp.dot(ub, wbT_ref[...], preferred_element_type=jnp.float32)
    acc = None
    for j in range(K):
        dj = (f_ref[0, j] - u).astype(jnp.bfloat16)
        aj = jnp.dot(dj, waT_ref[...], preferred_element_type=jnp.float32)
        acc = aj if acc is None else jnp.maximum(acc, aj)
    return _lrelu((acc + bterm) * INV * gm_ref[...] + bt_ref[...])


def _sel1_body(u_ref, idx_ref):
    idx_ref[0] = _topk_idx(u_ref[0], pl.program_id(0))


def _sel1(u, interpret=False):
    C = u.shape[2]
    return pl.pallas_call(
        _sel1_body,
        grid=(B,),
        in_specs=[pl.BlockSpec((1, N, C), lambda b: (b, 0, 0))],
        out_specs=pl.BlockSpec((1, N, K), lambda b: (b, 0, 0)),
        out_shape=jax.ShapeDtypeStruct((B, N, K), jnp.int32),
        interpret=interpret,
    )(u)


def _sel_mid_body(f_ref, u_ref, waT_ref, wbT_ref, gm_ref, bt_ref,
                  u_out, idx_ref):
    unew = _edge_conv(f_ref, u_ref[0], waT_ref, wbT_ref, gm_ref, bt_ref)
    u_out[0] = unew
    idx_ref[0] = _topk_idx(unew, pl.program_id(0))


def _sel_mid(f, u, waT, wbT, gm, bt, interpret=False):
    C = u.shape[2]
    D = waT.shape[1]
    blk = lambda c: pl.BlockSpec((1, N, c), lambda b: (b, 0, 0))
    return pl.pallas_call(
        _sel_mid_body,
        grid=(B,),
        in_specs=[pl.BlockSpec((1, K, N, C), lambda b: (b, 0, 0, 0)), blk(C),
                  pl.BlockSpec((C, D), lambda b: (0, 0)),
                  pl.BlockSpec((C, D), lambda b: (0, 0)),
                  pl.BlockSpec((D,), lambda b: (0,)),
                  pl.BlockSpec((D,), lambda b: (0,))],
        out_specs=[blk(D), pl.BlockSpec((1, N, K), lambda b: (b, 0, 0))],
        out_shape=[jax.ShapeDtypeStruct((B, N, D), jnp.float32),
                   jax.ShapeDtypeStruct((B, N, K), jnp.int32)],
        interpret=interpret,
    )(f, u, waT, wbT, gm, bt)


@functools.lru_cache(maxsize=None)
def _gath_sc(C):
    mesh = plsc.VectorSubcoreMesh(core_axis_name="c", subcore_axis_name="s")

    @functools.partial(
        pl.kernel,
        out_type=jax.ShapeDtypeStruct((B * K, N, C), jnp.float32),
        mesh=mesh,
        scratch_types=[
            pltpu.VMEM((PPW * K,), jnp.int32),
            pltpu.VMEM((PPW * K,), jnp.int32),
            pltpu.VMEM((PPW, C), jnp.float32),
            pltpu.VMEM((PPW, C), jnp.float32),
            pltpu.SemaphoreType.DMA,
            pltpu.SemaphoreType.DMA,
            pltpu.SemaphoreType.DMA,
            pltpu.SemaphoreType.DMA,
        ],
        compiler_params=pltpu.CompilerParams(use_tc_tiling_on_sc=False,
                                             needs_layout_passes=False),
    )
    def kern(u_hbm, idx_hbm, out_hbm, idx_v, idxT_v, buf0, buf1,
             gsem0, gsem1, wsem0, wsem1):
        wid = lax.axis_index("s") * NC + lax.axis_index("c")
        b = wid // WPB
        n0 = (wid % WPB) * PPW
        pltpu.sync_copy(idx_hbm.at[pl.ds(wid * PPW * K, PPW * K)], idx_v)
        iota = lax.iota(jnp.int32, 16)
        for j in range(K):
            for t in range(PPW // 16):
                iv = iota * K + (t * 16 * K + j)
                g = plsc.load_gather(idx_v, [iv])
                idxT_v[pl.ds(j * PPW + t * 16, 16)] = g
        bufs = (buf0, buf1)
        gsems = (gsem0, gsem1)
        wsems = (wsem0, wsem1)
        dummy = u_hbm.at[pl.ds(0, PPW)]

        def fire(j, d):
            pltpu.async_copy(
                u_hbm.at[idxT_v.at[pl.ds(j * PPW, PPW // 2)]],
                bufs[d].at[pl.ds(0, PPW // 2)], gsems[d])
            pltpu.async_copy(
                u_hbm.at[idxT_v.at[pl.ds(j * PPW + PPW // 2, PPW // 2)]],
                bufs[d].at[pl.ds(PPW // 2, PPW // 2)], gsems[d])

        fire(0, 0)
        for j in range(K):
            d = j % 2
            e = 1 - d
            pltpu.make_async_copy(dummy, bufs[d], gsems[d]).wait()
            pltpu.async_copy(bufs[d], out_hbm.at[b * K + j, pl.ds(n0, PPW)],
                             wsems[d])
            if j + 1 < K:
                if j >= 1:
                    pltpu.make_async_copy(dummy, bufs[e], wsems[e]).wait()
                fire(j + 1, e)
        pltpu.make_async_copy(dummy, bufs[(K - 1) % 2], wsems[(K - 1) % 2]).wait()
        pltpu.make_async_copy(dummy, bufs[(K - 2) % 2], wsems[(K - 2) % 2]).wait()

    return kern


def _pool_body(f4_ref, u3_ref, wa4T_ref, wb4T_ref, gm4_ref, bt4_ref,
               x1_ref, x2_ref, w5a_ref, w5b_ref, w5c_ref, w5d_ref,
               gm5_ref, bt5_ref, out_ref):
    x4 = _edge_conv(f4_ref, u3_ref[0], wa4T_ref, wb4T_ref, gm4_ref, bt4_ref)
    h = jnp.dot(x1_ref[0].astype(jnp.bfloat16), w5a_ref[...],
                preferred_element_type=jnp.float32)
    h += jnp.dot(x2_ref[0].astype(jnp.bfloat16), w5b_ref[...],
                 preferred_element_type=jnp.float32)
    h += jnp.dot(u3_ref[0].astype(jnp.bfloat16), w5c_ref[...],
                 preferred_element_type=jnp.float32)
    h += jnp.dot(x4.astype(jnp.bfloat16), w5d_ref[...],
                 preferred_element_type=jnp.float32)
    h = _lrelu(h * INV * gm5_ref[...] + bt5_ref[...])
    out_ref[0, 0] = jnp.max(h, axis=0)
    out_ref[0, 1] = jnp.sum(h, axis=0) * (1.0 / N)


def _pool(f4, u3, wa4T, wb4T, gm4, bt4, x1, x2, w5a, w5b, w5c, w5d, gm5, bt5,
          interpret=False):
    blk = lambda c: pl.BlockSpec((1, N, c), lambda b: (b, 0, 0))
    full2 = lambda a: pl.BlockSpec(a.shape, lambda b: (0, 0))
    vec = lambda c: pl.BlockSpec((c,), lambda b: (0,))
    return pl.pallas_call(
        _pool_body,
        grid=(B,),
        in_specs=[pl.BlockSpec((1, K, N, 128), lambda b: (b, 0, 0, 0)),
                  blk(128), full2(wa4T), full2(wb4T), vec(256), vec(256),
                  blk(64), blk(64), full2(w5a), full2(w5b), full2(w5c),
                  full2(w5d), vec(1024), vec(1024)],
        out_specs=pl.BlockSpec((1, 2, 1024), lambda b: (b, 0, 0)),
        out_shape=jax.ShapeDtypeStruct((B, 2, 1024), jnp.float32),
        interpret=interpret,
    )(f4, u3, wa4T, wb4T, gm4, bt4, x1, x2, w5a, w5b, w5c, w5d, gm5,
      bt5).reshape(B, 2048)


def _head_body(h_ref, l1_ref, g6_ref, b6_ref, l2_ref, lb2_ref, g7_ref,
               b7_ref, l3_ref, lb3_ref, out_ref):
    h = h_ref[...]
    a = jnp.dot(h.astype(jnp.bfloat16), l1_ref[...],
                preferred_element_type=jnp.float32)
    a = _lrelu(a * INV * g6_ref[...][None, :] + b6_ref[...][None, :])
    bb = jnp.dot(a.astype(jnp.bfloat16), l2_ref[...],
                 preferred_element_type=jnp.float32)
    bb = bb + lb2_ref[...][None, :]
    bb = _lrelu(bb * INV * g7_ref[...][None, :] + b7_ref[...][None, :])
    c = jnp.dot(bb.astype(jnp.bfloat16), l3_ref[...],
                preferred_element_type=jnp.float32)
    out_ref[...] = c + lb3_ref[...][None, :]


def _head(h, l1T, g6, b6, l2T, Lb2, g7, b7, l3T, Lb3, interpret=False):
    return pl.pallas_call(
        _head_body,
        out_shape=jax.ShapeDtypeStruct((h.shape[0], l3T.shape[1]),
                                       jnp.float32),
        interpret=interpret,
    )(h, l1T, g6, b6, l2T, Lb2, g7, b7, l3T, Lb3)


def kernel(x, W1, g1, b1, W2, g2, b2, W3, g3, b3, W4, g4, b4, W5, g5, b5,
           L1, g6, b6, L2, Lb2, g7, b7, L3, Lb3):
    bf = jnp.bfloat16
    xt0 = jnp.transpose(x, (0, 2, 1))
    xt0 = jnp.concatenate(
        [xt0, jnp.zeros((B, N, 13), jnp.float32)], axis=2)
    W1p = jnp.concatenate([W1[:, :3], jnp.zeros((64, 13), jnp.float32),
                           W1[:, 3:], jnp.zeros((64, 13), jnp.float32)],
                          axis=1)
    wts = []
    for W, c in ((W1p, 16), (W2, 64), (W3, 64), (W4, 128)):
        wts.append((W[:, :c].T.astype(bf), W[:, c:].T.astype(bf)))

    idx1 = _sel1(xt0)
    F1 = _gath_sc(16)(xt0.reshape(B * N, 16),
                      idx1.reshape(-1)).reshape(B, K, N, 16)
    u1, idx2 = _sel_mid(F1, xt0, *wts[0], g1, b1)
    F2 = _gath_sc(64)(u1.reshape(B * N, 64),
                      idx2.reshape(-1)).reshape(B, K, N, 64)
    u2, idx3 = _sel_mid(F2, u1, *wts[1], g2, b2)
    F3 = _gath_sc(64)(u2.reshape(B * N, 64),
                      idx3.reshape(-1)).reshape(B, K, N, 64)
    u3, idx4 = _sel_mid(F3, u2, *wts[2], g3, b3)
    F4 = _gath_sc(128)(u3.reshape(B * N, 128),
                       idx4.reshape(-1)).reshape(B, K, N, 128)

    w5 = [W5[:, a:b].T.astype(bf)
          for a, b in ((0, 64), (64, 128), (128, 256), (256, 512))]
    pooled = _pool(F4, u3, *wts[3], g4, b4, u1, u2, *w5, g5, b5)
    return _head(pooled, L1.T.astype(bf), g6, b6, L2.T.astype(bf), Lb2,
                 g7, b7, L3.T.astype(bf), Lb3)
